# baseline (speedup 1.0000x reference)
import functools
import math

import jax
import jax.numpy as jnp
import numpy as np
from jax.experimental import pallas as pl

_B, _T, _C = 4, 2048, 1024
_RR = 0.95
_MARGIN = 0.5
_R = math.floor(_T - _T * _RR)
_TWO_R = 2 * _R
_KEEP = _T - _TWO_R
_MT = 128


def _colsum_kernel(xn_tile_ref, xn_full_ref, nstd_ref, diag_ref, out_ref):
    i = pl.program_id(1)

    @pl.when(i == 0)
    def _():
        out_ref[...] = jnp.zeros_like(out_ref)

    xt = xn_tile_ref[0]
    xf = xn_full_ref[0]
    score = jax.lax.dot_general(
        xt, xf, (((1,), (1,)), ((), ())),
        preferred_element_type=jnp.float32,
        precision=jax.lax.Precision.DEFAULT,
    )
    nstd = nstd_ref[0]
    dg = diag_ref[0]
    ii = jax.lax.broadcasted_iota(jnp.int32, score.shape, 0)
    jj = jax.lax.broadcasted_iota(jnp.int32, score.shape, 1)
    on_diag = jj == (ii + i * _MT)
    sel = jnp.where(score > _MARGIN, score + (-_MARGIN),
                    jnp.broadcast_to(nstd, score.shape))
    sel = jnp.where(on_diag, jnp.broadcast_to(dg + (-_MARGIN), score.shape),
                    sel)
    acc = sel[0:8, :]
    for j in range(1, _MT // 8):
        acc = acc + sel[j * 8:(j + 1) * 8, :]
    v4 = acc[0:4, :] + acc[4:8, :]
    v2 = v4[0:2, :] + v4[2:4, :]
    v1 = v2[0:1, :] + v2[1:2, :]
    out_ref[0] += v1


def _colsum(xn, neg_std_rows, diag_rows):
    return pl.pallas_call(
        _colsum_kernel,
        grid=(_B, _T // _MT),
        in_specs=[
            pl.BlockSpec((1, _MT, _C), lambda b, i: (b, i, 0)),
            pl.BlockSpec((1, _T, _C), lambda b, i: (b, 0, 0)),
            pl.BlockSpec((1, _MT, 1), lambda b, i: (b, i, 0)),
            pl.BlockSpec((1, _MT, 1), lambda b, i: (b, i, 0)),
        ],
        out_specs=pl.BlockSpec((1, 1, _T), lambda b, i: (b, 0, 0)),
        out_shape=jax.ShapeDtypeStruct((_B, 1, _T), jnp.float32),
    )(xn, xn, neg_std_rows, diag_rows)[:, 0]


def kernel(x):
    xn = x / jnp.clip(jnp.linalg.norm(x, axis=-1, keepdims=True), 1e-12, None)
    x_std = jnp.std(xn, axis=-1, ddof=1, keepdims=True)
    neg_std = -1.0 * x_std

    sc_aux = jnp.einsum('btc,bsc->bts', xn, xn)
    diag = jnp.diagonal(sc_aux, axis1=1, axis2=2)[..., None]

    col_sum = _colsum(xn, neg_std, diag)
    col_mean = col_sum * np.float32(1.0 / _T)

    min_indices = jnp.argsort(-col_mean, axis=-1)[..., :_TWO_R]
    a_idx = min_indices[..., ::2]
    b_idx = min_indices[..., 1::2]
    a = jnp.take_along_axis(xn, a_idx[..., None], axis=1)
    b = jnp.take_along_axis(xn, b_idx[..., None], axis=1)
    scores = jnp.einsum('brc,bsc->brs', a, b)
    dst_idx = jnp.argmax(scores, axis=-1)

    batch = jnp.arange(_B)[:, None]
    keep = jnp.ones((_B, _T), dtype=bool).at[batch, min_indices].set(False)
    order = jnp.argsort(jnp.where(keep, 0, 1).astype(jnp.int32),
                        axis=-1)[:, :_KEEP]
    ori = jnp.take_along_axis(x, order[..., None], axis=1)
    src = jnp.take_along_axis(x, a_idx[..., None], axis=1)
    dst = jnp.take_along_axis(x, b_idx[..., None], axis=1)
    counts = jnp.ones((_B, _R), dtype=x.dtype).at[batch, dst_idx].add(1.0)
    dst = dst.at[batch, dst_idx].add(src)
    dst = dst / counts[..., None]
    return jnp.concatenate([ori, dst], axis=1)
